# baseline (speedup 1.0000x reference)
import functools

import jax
import jax.numpy as jnp
from jax import lax
from jax.experimental import pallas as pl
from jax.experimental.pallas import tpu as pltpu
from jax.experimental.pallas import tpu_sc as plsc

_FEATURE_DIM = 64
_NUM_CLASSES = 11
_C_PAD = 16
_ALPHA = 0.9

_VC_TC = 16384
_VS = 512
_NW = 32
_ROWS = 2 * _C_PAD
_COLS = 80
_BANK = _ROWS * _COLS + 1
_NLANE = 16


def _tc_seg_body(lab_ref, emb_ref, out_ref, acc_ref, cnt_ref):
    i = pl.program_id(0)
    nc = pl.num_programs(0)

    @pl.when(i == 0)
    def _init():
        acc_ref[...] = jnp.zeros_like(acc_ref)
        cnt_ref[...] = jnp.zeros_like(cnt_ref)

    classes = lax.broadcasted_iota(jnp.int32, (_C_PAD, 1), 0)
    B = emb_ref.shape[0]
    for b in range(B):
        lab = lab_ref[0, b, :]
        onehot = (lab[None, :] == classes).astype(jnp.float32)
        part = lax.dot_general(
            onehot, emb_ref[b], (((1,), (1,)), ((), ())),
            preferred_element_type=jnp.float32)
        acc_ref[b] += part
        cnt_ref[b] += jnp.sum(onehot, axis=1, keepdims=True)

    @pl.when(i == nc - 1)
    def _finalize():
        pad = jnp.zeros((_C_PAD, _COLS - _FEATURE_DIM - 1), jnp.float32)
        for b in range(B):
            out_ref[b] = jnp.concatenate([acc_ref[b], cnt_ref[b], pad], axis=1)


def _sc_body(n_sc_chunks, vtc, v_total, emb_hbm, lab_hbm, out_hbm,
             e0, e1, l0, l1, accv, packv,
             se0, se1, sl0, sl1, so):
    nc2 = plsc.get_sparse_core_info().num_cores
    wid = lax.axis_index("s") * nc2 + lax.axis_index("c")
    vw = n_sc_chunks * _VS
    base = vtc + wid * vw

    def _zero(i, _):
        accv[pl.ds(i * 16, 16)] = jnp.zeros((16,), jnp.float32)
        return 0
    lax.fori_loop(0, (_NLANE * _BANK) // 16, _zero, 0, unroll=8)

    lanebank = lax.iota(jnp.int32, 16) * _BANK
    ones16 = jnp.ones((16,), jnp.float32)

    ebuf = (e0, e1)
    lbuf = (l0, l1)
    esem = (se0, se1)
    lsem = (sl0, sl1)
    T = 2 * n_sc_chunks

    def start(t):
        b, ci = divmod(t, n_sc_chunks)
        s = base + ci * _VS
        sl = t % 2
        ce = pltpu.async_copy(emb_hbm.at[b, :, pl.ds(s, _VS)], ebuf[sl], esem[sl])
        cl = pltpu.async_copy(lab_hbm.at[b, pl.ds(s, _VS)], lbuf[sl], lsem[sl])
        return ce, cl

    cps = {0: start(0)}
    for t in range(T):
        if t + 1 < T:
            cps[t + 1] = start(t + 1)
        ce, cl = cps.pop(t)
        ce.wait()
        cl.wait()
        b = t // n_sc_chunks
        sl = t % 2
        eb = ebuf[sl]
        lb = lbuf[sl]

        def group(j, _, b=b, eb=eb, lb=lb):
            lab16 = lb[pl.ds(j * 16, 16)]
            row = lab16 + (b * _C_PAD)
            idx = row * _COLS + lanebank
            for f in range(_FEATURE_DIM):
                v = eb[f, pl.ds(j * 16, 16)]
                plsc.addupdate_scatter(accv, [idx + f], v)
            plsc.addupdate_scatter(accv, [idx + _FEATURE_DIM], ones16)
            return 0

        lax.fori_loop(0, _VS // 16, group, 0, unroll=2)

    def pack_row(r, _):
        for g in range(_COLS // 16):
            o = r * _COLS + g * 16
            val = accv[pl.ds(o, 16)]
            for bank in range(1, _NLANE):
                val = val + accv[pl.ds(bank * _BANK + o, 16)]
            packv[r, pl.ds(g * 16, 16)] = val
        return 0
    lax.fori_loop(0, _ROWS, pack_row, 0)

    pltpu.async_copy(packv, out_hbm.at[wid], so).wait()


def _combine_body(tc_ref, sc_ref, out_ref):
    red = jnp.sum(sc_ref[...], axis=0)
    tc = tc_ref[...]
    s0 = tc[0, :, :_FEATURE_DIM] + red[:_C_PAD, :_FEATURE_DIM]
    s1 = tc[1, :, :_FEATURE_DIM] + red[_C_PAD:, :_FEATURE_DIM]
    c0 = tc[0, :, _FEATURE_DIM:_FEATURE_DIM + 1] + red[:_C_PAD, _FEATURE_DIM:_FEATURE_DIM + 1]
    c1 = tc[1, :, _FEATURE_DIM:_FEATURE_DIM + 1] + red[_C_PAD:, _FEATURE_DIM:_FEATURE_DIM + 1]
    m0 = s0 / jnp.maximum(c0, 1.0)
    m1 = s1 / jnp.maximum(c1, 1.0)
    p0 = c0 > 0.0
    p1 = c1 > 0.0
    upd = jnp.where(p0, _ALPHA * m0 + (1.0 - _ALPHA) * m1, m1)
    out_ref[...] = jnp.where(p1, upd, jnp.where(p0, m0, 0.0))


def kernel(embeddings, labels):
    B, F, D, H, W = embeddings.shape
    V = D * H * W
    emb3 = embeddings.reshape(B, F, V)
    lab2 = labels.reshape(B, V).astype(jnp.int32)

    n_chunks = V // _VC_TC
    k_sc = 6
    n_tc = n_chunks - k_sc
    vtc = n_tc * _VC_TC
    n_sc_chunks = (k_sc * _VC_TC) // (_NW * _VS)

    lab_r = lab2.reshape(B, n_chunks, _VC_TC).transpose(1, 0, 2)

    sc_kernel = functools.partial(
        pl.kernel,
        out_type=jax.ShapeDtypeStruct((_NW, _ROWS, _COLS), jnp.float32),
        mesh=plsc.VectorSubcoreMesh(core_axis_name="c", subcore_axis_name="s"),
        scratch_types=[
            pltpu.VMEM((F, _VS), jnp.float32),
            pltpu.VMEM((F, _VS), jnp.float32),
            pltpu.VMEM((_VS,), jnp.int32),
            pltpu.VMEM((_VS,), jnp.int32),
            pltpu.VMEM((_NLANE * _BANK,), jnp.float32),
            pltpu.VMEM((_ROWS, _COLS), jnp.float32),
            pltpu.SemaphoreType.DMA,
            pltpu.SemaphoreType.DMA,
            pltpu.SemaphoreType.DMA,
            pltpu.SemaphoreType.DMA,
            pltpu.SemaphoreType.DMA,
        ],
        compiler_params=pltpu.CompilerParams(
            needs_layout_passes=False, disable_bounds_checks=True),
        cost_estimate=pl.CostEstimate(
            flops=2 * B * k_sc * _VC_TC * F,
            transcendentals=0,
            bytes_accessed=B * k_sc * _VC_TC * (F + 1) * 4,
        ),
    )(functools.partial(_sc_body, n_sc_chunks, vtc, V))
    sc_part = sc_kernel(emb3, lab2)

    tc_part = pl.pallas_call(
        _tc_seg_body,
        grid=(n_tc,),
        in_specs=[
            pl.BlockSpec((1, B, _VC_TC), lambda i: (i, 0, 0)),
            pl.BlockSpec((B, F, _VC_TC), lambda i: (0, 0, i)),
        ],
        out_specs=pl.BlockSpec((B, _C_PAD, _COLS), lambda i: (0, 0, 0)),
        out_shape=jax.ShapeDtypeStruct((B, _C_PAD, _COLS), jnp.float32),
        scratch_shapes=[
            pltpu.VMEM((B, _C_PAD, F), jnp.float32),
            pltpu.VMEM((B, _C_PAD, 1), jnp.float32),
        ],
        cost_estimate=pl.CostEstimate(
            flops=2 * B * n_tc * _VC_TC * (_C_PAD * F),
            transcendentals=0,
            bytes_accessed=B * n_tc * _VC_TC * (F + 1) * 4,
        ),
    )(lab_r, emb3)

    protos = pl.pallas_call(
        _combine_body,
        in_specs=[
            pl.BlockSpec((B, _C_PAD, _COLS), lambda: (0, 0, 0)),
            pl.BlockSpec((_NW, _ROWS, _COLS), lambda: (0, 0, 0)),
        ],
        out_specs=pl.BlockSpec((_C_PAD, F), lambda: (0, 0)),
        out_shape=jax.ShapeDtypeStruct((_C_PAD, F), jnp.float32),
    )(tc_part, sc_part)
    return protos[:_NUM_CLASSES]


# speedup vs baseline: 1.0562x; 1.0562x over previous
import functools

import jax
import jax.numpy as jnp
from jax import lax
from jax.experimental import pallas as pl
from jax.experimental.pallas import tpu as pltpu
from jax.experimental.pallas import tpu_sc as plsc

_FEATURE_DIM = 64
_NUM_CLASSES = 11
_C_PAD = 16
_ALPHA = 0.9

_VC_TC = 16384
_VS = 512
_NW = 32
_ROWS = 2 * _C_PAD
_COLS = 80
_BANK = _ROWS * _COLS + 1
_NLANE = 16


def _tc_seg_body(lab_ref, emb_ref, out_ref, acc_ref, cnt_ref):
    i = pl.program_id(0)
    nc = pl.num_programs(0)

    @pl.when(i == 0)
    def _init():
        acc_ref[...] = jnp.zeros_like(acc_ref)
        cnt_ref[...] = jnp.zeros_like(cnt_ref)

    classes = lax.broadcasted_iota(jnp.int32, (_C_PAD, 1), 0)
    B = emb_ref.shape[0]
    for b in range(B):
        lab = lab_ref[0, b, :]
        onehot = (lab[None, :] == classes).astype(jnp.float32)
        part = lax.dot_general(
            onehot, emb_ref[b], (((1,), (1,)), ((), ())),
            preferred_element_type=jnp.float32)
        acc_ref[b] += part
        cnt_ref[b] += jnp.sum(onehot, axis=1, keepdims=True)

    @pl.when(i == nc - 1)
    def _finalize():
        pad = jnp.zeros((_C_PAD, _COLS - _FEATURE_DIM - 1), jnp.float32)
        for b in range(B):
            out_ref[b] = jnp.concatenate([acc_ref[b], cnt_ref[b], pad], axis=1)


def _sc_body(n_sc_chunks, vtc, v_total, emb_hbm, lab_hbm, out_hbm,
             e0, e1, l0, l1, accv, packv,
             se0, se1, sl0, sl1, so):
    nc2 = plsc.get_sparse_core_info().num_cores
    wid = lax.axis_index("s") * nc2 + lax.axis_index("c")
    vw = n_sc_chunks * _VS
    base = vtc + wid * vw

    def _zero(i, _):
        accv[pl.ds(i * 16, 16)] = jnp.zeros((16,), jnp.float32)
        return 0
    lax.fori_loop(0, (_NLANE * _BANK) // 16, _zero, 0, unroll=8)

    lanebank = lax.iota(jnp.int32, 16) * _BANK
    ones16 = jnp.ones((16,), jnp.float32)

    ebuf = (e0, e1)
    lbuf = (l0, l1)
    esem = (se0, se1)
    lsem = (sl0, sl1)
    T = 2 * n_sc_chunks

    def start(t):
        b, ci = divmod(t, n_sc_chunks)
        s = base + ci * _VS
        sl = t % 2
        ce = pltpu.async_copy(emb_hbm.at[b, :, pl.ds(s, _VS)], ebuf[sl], esem[sl])
        cl = pltpu.async_copy(lab_hbm.at[b, pl.ds(s, _VS)], lbuf[sl], lsem[sl])
        return ce, cl

    cps = {0: start(0)}
    for t in range(T):
        if t + 1 < T:
            cps[t + 1] = start(t + 1)
        ce, cl = cps.pop(t)
        ce.wait()
        cl.wait()
        b = t // n_sc_chunks
        sl = t % 2
        eb = ebuf[sl]
        lb = lbuf[sl]

        def group(j, _, b=b, eb=eb, lb=lb):
            lab16 = lb[pl.ds(j * 16, 16)]
            row = lab16 + (b * _C_PAD)
            idx = row * _COLS + lanebank
            for f in range(_FEATURE_DIM):
                v = eb[f, pl.ds(j * 16, 16)]
                plsc.addupdate_scatter(accv, [idx + f], v)
            plsc.addupdate_scatter(accv, [idx + _FEATURE_DIM], ones16)
            return 0

        lax.fori_loop(0, _VS // 16, group, 0, unroll=2)

    def pack_row(r, _):
        for g in range(_COLS // 16):
            o = r * _COLS + g * 16
            val = accv[pl.ds(o, 16)]
            for bank in range(1, _NLANE):
                val = val + accv[pl.ds(bank * _BANK + o, 16)]
            packv[r, pl.ds(g * 16, 16)] = val
        return 0
    lax.fori_loop(0, _ROWS, pack_row, 0)

    pltpu.async_copy(packv, out_hbm.at[wid], so).wait()


def _combine_body(tc_ref, sc_ref, out_ref):
    red = jnp.sum(sc_ref[...], axis=0)
    tc = tc_ref[...]
    s0 = tc[0, :, :_FEATURE_DIM] + red[:_C_PAD, :_FEATURE_DIM]
    s1 = tc[1, :, :_FEATURE_DIM] + red[_C_PAD:, :_FEATURE_DIM]
    c0 = tc[0, :, _FEATURE_DIM:_FEATURE_DIM + 1] + red[:_C_PAD, _FEATURE_DIM:_FEATURE_DIM + 1]
    c1 = tc[1, :, _FEATURE_DIM:_FEATURE_DIM + 1] + red[_C_PAD:, _FEATURE_DIM:_FEATURE_DIM + 1]
    m0 = s0 / jnp.maximum(c0, 1.0)
    m1 = s1 / jnp.maximum(c1, 1.0)
    p0 = c0 > 0.0
    p1 = c1 > 0.0
    upd = jnp.where(p0, _ALPHA * m0 + (1.0 - _ALPHA) * m1, m1)
    out_ref[...] = jnp.where(p1, upd, jnp.where(p0, m0, 0.0))


def kernel(embeddings, labels):
    B, F, D, H, W = embeddings.shape
    V = D * H * W
    emb3 = embeddings.reshape(B, F, V)
    lab2 = labels.reshape(B, V).astype(jnp.int32)

    n_chunks = V // _VC_TC
    k_sc = 2
    n_tc = n_chunks - k_sc
    vtc = n_tc * _VC_TC
    n_sc_chunks = (k_sc * _VC_TC) // (_NW * _VS)

    lab_r = lab2.reshape(B, n_chunks, _VC_TC).transpose(1, 0, 2)

    sc_kernel = functools.partial(
        pl.kernel,
        out_type=jax.ShapeDtypeStruct((_NW, _ROWS, _COLS), jnp.float32),
        mesh=plsc.VectorSubcoreMesh(core_axis_name="c", subcore_axis_name="s"),
        scratch_types=[
            pltpu.VMEM((F, _VS), jnp.float32),
            pltpu.VMEM((F, _VS), jnp.float32),
            pltpu.VMEM((_VS,), jnp.int32),
            pltpu.VMEM((_VS,), jnp.int32),
            pltpu.VMEM((_NLANE * _BANK,), jnp.float32),
            pltpu.VMEM((_ROWS, _COLS), jnp.float32),
            pltpu.SemaphoreType.DMA,
            pltpu.SemaphoreType.DMA,
            pltpu.SemaphoreType.DMA,
            pltpu.SemaphoreType.DMA,
            pltpu.SemaphoreType.DMA,
        ],
        compiler_params=pltpu.CompilerParams(
            needs_layout_passes=False, disable_bounds_checks=True),
        cost_estimate=pl.CostEstimate(
            flops=2 * B * k_sc * _VC_TC * F,
            transcendentals=0,
            bytes_accessed=B * k_sc * _VC_TC * (F + 1) * 4,
        ),
    )(functools.partial(_sc_body, n_sc_chunks, vtc, V))
    sc_part = sc_kernel(emb3, lab2)

    tc_part = pl.pallas_call(
        _tc_seg_body,
        grid=(n_tc,),
        in_specs=[
            pl.BlockSpec((1, B, _VC_TC), lambda i: (i, 0, 0)),
            pl.BlockSpec((B, F, _VC_TC), lambda i: (0, 0, i)),
        ],
        out_specs=pl.BlockSpec((B, _C_PAD, _COLS), lambda i: (0, 0, 0)),
        out_shape=jax.ShapeDtypeStruct((B, _C_PAD, _COLS), jnp.float32),
        scratch_shapes=[
            pltpu.VMEM((B, _C_PAD, F), jnp.float32),
            pltpu.VMEM((B, _C_PAD, 1), jnp.float32),
        ],
        cost_estimate=pl.CostEstimate(
            flops=2 * B * n_tc * _VC_TC * (_C_PAD * F),
            transcendentals=0,
            bytes_accessed=B * n_tc * _VC_TC * (F + 1) * 4,
        ),
    )(lab_r, emb3)

    protos = pl.pallas_call(
        _combine_body,
        in_specs=[
            pl.BlockSpec((B, _C_PAD, _COLS), lambda: (0, 0, 0)),
            pl.BlockSpec((_NW, _ROWS, _COLS), lambda: (0, 0, 0)),
        ],
        out_specs=pl.BlockSpec((_C_PAD, F), lambda: (0, 0)),
        out_shape=jax.ShapeDtypeStruct((_C_PAD, F), jnp.float32),
    )(tc_part, sc_part)
    return protos[:_NUM_CLASSES]
